# baseline (speedup 1.0000x reference)
import functools

import jax
import jax.numpy as jnp
from jax.experimental import pallas as pl
from jax.experimental.pallas import tpu as pltpu

_VMEM_LIMIT = 48 * 1024 * 1024


def _bn_params(st_ref, g_ref, b_ref, count, eps):
    s = jnp.sum(st_ref[...], axis=0)
    mu = s[0] / count
    var = s[1] / count - mu * mu
    scale = g_ref[...] * jax.lax.rsqrt(var + eps)
    shift = b_ref[...] - mu * scale
    return scale, shift


def _conv_kernel(x_ref, w_ref, st_ref, g_ref, b_ref, y_ref, stats_ref, *,
                 cin, cout, width, apply_bn_relu, count, eps):
    B, _, M = x_ref.shape
    x = x_ref[...]
    if apply_bn_relu:
        scale, shift = _bn_params(st_ref, g_ref, b_ref, count, eps)
        x = jnp.maximum(x.astype(jnp.float32) * scale + shift,
                        0.0).astype(jnp.bfloat16)
    else:
        x = x.astype(jnp.bfloat16)

    xa = jnp.concatenate([x[b] for b in range(B)], axis=1)

    col = jax.lax.broadcasted_iota(jnp.int32, (1, B * M), 1) % width
    zc = jnp.zeros((cin, 1), jnp.bfloat16)
    x_l = jnp.where(col == 0, jnp.bfloat16(0),
                    jnp.concatenate([zc, xa[:, :-1]], axis=1))
    x_r = jnp.where(col == width - 1, jnp.bfloat16(0),
                    jnp.concatenate([xa[:, 1:], zc], axis=1))

    x3 = jnp.concatenate([x_l, xa, x_r], axis=0)
    z = jnp.dot(w_ref[...], x3, preferred_element_type=jnp.float32)

    acc = jnp.zeros((cout, M), jnp.float32)
    acc2 = jnp.zeros((cout, M), jnp.float32)
    zrow = jnp.zeros((cout, width), jnp.float32)
    for b in range(B):
        lo = b * M
        z0 = z[:cout, lo:lo + M]
        z1 = z[cout:2 * cout, lo:lo + M]
        z2 = z[2 * cout:, lo:lo + M]
        y_b = (z1 + jnp.concatenate([zrow, z0[:, :-width]], axis=1)
                  + jnp.concatenate([z2[:, width:], zrow], axis=1))
        y_ref[b] = y_b.astype(y_ref.dtype)
        acc = acc + y_b
        acc2 = acc2 + y_b * y_b

    cs = jnp.sum(acc, axis=1, keepdims=True)
    css = jnp.sum(acc2, axis=1, keepdims=True)
    stats_ref[...] = jnp.stack([cs, css], axis=0)[None]


def _conv(x, wmat, st, g, b, *, width, apply_bn_relu, out_dtype, block_n,
          count, eps):
    N, cin, M = x.shape
    cout = wmat.shape[0] // 3
    grid = N // block_n
    body = functools.partial(_conv_kernel, cin=cin, cout=cout, width=width,
                             apply_bn_relu=apply_bn_relu, count=count, eps=eps)
    return pl.pallas_call(
        body,
        out_shape=(jax.ShapeDtypeStruct((N, cout, M), out_dtype),
                   jax.ShapeDtypeStruct((grid, 2, cout, 1), jnp.float32)),
        grid=(grid,),
        in_specs=[
            pl.BlockSpec((block_n, cin, M), lambda n: (n, 0, 0)),
            pl.BlockSpec(wmat.shape, lambda n: (0, 0)),
            pl.BlockSpec(st.shape, lambda n: (0, 0, 0, 0)),
            pl.BlockSpec(g.shape, lambda n: (0, 0)),
            pl.BlockSpec(b.shape, lambda n: (0, 0)),
        ],
        out_specs=(pl.BlockSpec((block_n, cout, M), lambda n: (n, 0, 0)),
                   pl.BlockSpec((1, 2, cout, 1), lambda n: (n, 0, 0, 0))),
        compiler_params=pltpu.CompilerParams(
            dimension_semantics=("parallel",),
            vmem_limit_bytes=_VMEM_LIMIT),
    )(x, wmat, st, g, b)


def _bn_relu_kernel(x_ref, st_ref, g_ref, b_ref, o_ref, *, count, eps):
    scale, shift = _bn_params(st_ref, g_ref, b_ref, count, eps)
    o_ref[...] = jnp.maximum(
        x_ref[...].astype(jnp.float32) * scale + shift, 0.0)


def _bn_relu(y, st, g, b, *, block_n, count, eps):
    N, c, M = y.shape
    grid = N // block_n
    body = functools.partial(_bn_relu_kernel, count=count, eps=eps)
    return pl.pallas_call(
        body,
        out_shape=jax.ShapeDtypeStruct((N, c, M), jnp.float32),
        grid=(grid,),
        in_specs=[
            pl.BlockSpec((block_n, c, M), lambda n: (n, 0, 0)),
            pl.BlockSpec(st.shape, lambda n: (0, 0, 0, 0)),
            pl.BlockSpec(g.shape, lambda n: (0, 0)),
            pl.BlockSpec(b.shape, lambda n: (0, 0)),
        ],
        out_specs=pl.BlockSpec((block_n, c, M), lambda n: (n, 0, 0)),
        compiler_params=pltpu.CompilerParams(
            dimension_semantics=("parallel",),
            vmem_limit_bytes=_VMEM_LIMIT),
    )(y, st, g, b)


def kernel(x_nchw, w1, g1, b1, w2, g2, b2, eps=1e-5):
    N, cin, H, W = x_nchw.shape
    c1 = w1.shape[-1]
    c2 = w2.shape[-1]
    M = H * W
    count = N * M
    block_n = 16

    x = x_nchw.reshape(N, cin, M)

    wm1 = jnp.transpose(w1, (0, 3, 1, 2)).reshape(3 * c1, 3 * cin)
    wm2 = jnp.transpose(w2, (0, 3, 1, 2)).reshape(3 * c2, 3 * c1)
    wm1 = wm1.astype(jnp.bfloat16)
    wm2 = wm2.astype(jnp.bfloat16)

    g1c = g1[:, None]
    b1c = b1[:, None]
    g2c = g2[:, None]
    b2c = b2[:, None]
    st0 = jnp.zeros((1, 2, cin, 1), jnp.float32)

    y1, st1 = _conv(x, wm1, st0, g1c, b1c, width=W, apply_bn_relu=False,
                    out_dtype=jnp.bfloat16, block_n=block_n,
                    count=count, eps=eps)

    y2, st2 = _conv(y1, wm2, st1, g1c, b1c, width=W, apply_bn_relu=True,
                    out_dtype=jnp.bfloat16, block_n=block_n,
                    count=count, eps=eps)

    a2 = _bn_relu(y2, st2, g2c, b2c, block_n=block_n, count=count, eps=eps)

    return a2.reshape(N, c2, H, W)


# speedup vs baseline: 1.0442x; 1.0442x over previous
import functools

import jax
import jax.numpy as jnp
from jax.experimental import pallas as pl
from jax.experimental.pallas import tpu as pltpu

_VMEM_LIMIT = 56 * 1024 * 1024


def _conv_block(x, w_ref, *, cin, cout, width, B, M):
    xa = jnp.concatenate([x[b] for b in range(B)], axis=1)

    col = jax.lax.broadcasted_iota(jnp.int32, (1, B * M), 1) % width
    zc = jnp.zeros((cin, 1), jnp.bfloat16)
    x_l = jnp.where(col == 0, jnp.bfloat16(0),
                    jnp.concatenate([zc, xa[:, :-1]], axis=1))
    x_r = jnp.where(col == width - 1, jnp.bfloat16(0),
                    jnp.concatenate([xa[:, 1:], zc], axis=1))

    x3 = jnp.concatenate([x_l, xa, x_r], axis=0)
    z = jnp.dot(w_ref[...], x3, preferred_element_type=jnp.float32)

    ys = []
    zrow = jnp.zeros((cout, width), jnp.float32)
    for b in range(B):
        lo = b * M
        z0 = z[:cout, lo:lo + M]
        z1 = z[cout:2 * cout, lo:lo + M]
        z2 = z[2 * cout:, lo:lo + M]
        ys.append(z1 + jnp.concatenate([zrow, z0[:, :-width]], axis=1)
                     + jnp.concatenate([z2[:, width:], zrow], axis=1))
    return ys


def _accum_stats(st_sc, ys, first):
    cs = sum(jnp.sum(y, axis=1, keepdims=True) for y in ys)
    css = sum(jnp.sum(y * y, axis=1, keepdims=True) for y in ys)
    s = jnp.stack([cs, css], axis=0)

    @pl.when(first)
    def _():
        st_sc[...] = s

    @pl.when(jnp.logical_not(first))
    def _():
        st_sc[...] = st_sc[...] + s


def _bn_params(st_sc, g_ref, b_ref, count, eps):
    s = st_sc[...]
    mu = s[0] / count
    var = s[1] / count - mu * mu
    scale = g_ref[...] * jax.lax.rsqrt(var + eps)
    shift = b_ref[...] - mu * scale
    return scale, shift


def _fused_kernel(x_ref, w1_ref, w2_ref, g1_ref, b1_ref, g2_ref, b2_ref,
                  o_ref, y_sc, st1_sc, st2_sc, *,
                  B, G, cin, c1, c2, width, M, count, eps):
    i = pl.program_id(0)

    @pl.when(i < G)
    def _phase_conv1():
        x = x_ref[...].astype(jnp.bfloat16)
        ys = _conv_block(x, w1_ref, cin=cin, cout=c1, width=width, B=B, M=M)
        for b in range(B):
            y_sc[i * B + b] = ys[b].astype(jnp.bfloat16)
        _accum_stats(st1_sc, ys, i == 0)

    @pl.when(jnp.logical_and(i >= G, i < 2 * G))
    def _phase_conv2():
        j = i - G
        scale1, shift1 = _bn_params(st1_sc, g1_ref, b1_ref, count, eps)
        a = y_sc[pl.ds(j * B, B)]
        a = jnp.maximum(a.astype(jnp.float32) * scale1 + shift1,
                        0.0).astype(jnp.bfloat16)
        ys = _conv_block(a, w2_ref, cin=c1, cout=c2, width=width, B=B, M=M)
        for b in range(B):
            y_sc[j * B + b] = ys[b].astype(jnp.bfloat16)
        _accum_stats(st2_sc, ys, j == 0)

    @pl.when(i >= 2 * G)
    def _phase_bn2():
        k = i - 2 * G
        scale2, shift2 = _bn_params(st2_sc, g2_ref, b2_ref, count, eps)
        y2 = y_sc[pl.ds(k * B, B)]
        o_ref[...] = jnp.maximum(
            y2.astype(jnp.float32) * scale2 + shift2, 0.0)


def kernel(x_nchw, w1, g1, b1, w2, g2, b2, eps=1e-5):
    N, cin, H, W = x_nchw.shape
    c1 = w1.shape[-1]
    c2 = w2.shape[-1]
    M = H * W
    count = N * M
    B = 16
    G = N // B

    x = x_nchw.reshape(N, cin, M)

    wm1 = jnp.transpose(w1, (0, 3, 1, 2)).reshape(3 * c1, 3 * cin)
    wm2 = jnp.transpose(w2, (0, 3, 1, 2)).reshape(3 * c2, 3 * c1)
    wm1 = wm1.astype(jnp.bfloat16)
    wm2 = wm2.astype(jnp.bfloat16)

    body = functools.partial(_fused_kernel, B=B, G=G, cin=cin, c1=c1, c2=c2,
                             width=W, M=M, count=count, eps=eps)
    out = pl.pallas_call(
        body,
        out_shape=jax.ShapeDtypeStruct((N, c2, M), jnp.float32),
        grid=(3 * G,),
        in_specs=[
            pl.BlockSpec((B, cin, M),
                         lambda i: (jnp.minimum(i, G - 1), 0, 0)),
            pl.BlockSpec(wm1.shape, lambda i: (0, 0)),
            pl.BlockSpec(wm2.shape, lambda i: (0, 0)),
            pl.BlockSpec((c1, 1), lambda i: (0, 0)),
            pl.BlockSpec((c1, 1), lambda i: (0, 0)),
            pl.BlockSpec((c2, 1), lambda i: (0, 0)),
            pl.BlockSpec((c2, 1), lambda i: (0, 0)),
        ],
        out_specs=pl.BlockSpec((B, c2, M),
                               lambda i: (jnp.maximum(i - 2 * G, 0), 0, 0)),
        scratch_shapes=[
            pltpu.VMEM((N, c1, M), jnp.bfloat16),
            pltpu.VMEM((2, c1, 1), jnp.float32),
            pltpu.VMEM((2, c2, 1), jnp.float32),
        ],
        compiler_params=pltpu.CompilerParams(
            dimension_semantics=("arbitrary",),
            vmem_limit_bytes=_VMEM_LIMIT),
    )(x, wm1, wm2, g1[:, None], b1[:, None], g2[:, None], b2[:, None])

    return out.reshape(N, c2, H, W)


# speedup vs baseline: 1.0902x; 1.0441x over previous
import functools

import jax
import jax.numpy as jnp
from jax.experimental import pallas as pl
from jax.experimental.pallas import tpu as pltpu

_VMEM_LIMIT = 56 * 1024 * 1024


def _conv_block(x, w_ref, *, cin, cout, width, B, M):
    xa = jnp.concatenate([x[b] for b in range(B)], axis=1)

    col = jax.lax.broadcasted_iota(jnp.int32, (1, B * M), 1) % width
    zc = jnp.zeros((cin, 1), jnp.bfloat16)
    x_l = jnp.where(col == 0, jnp.bfloat16(0),
                    jnp.concatenate([zc, xa[:, :-1]], axis=1))
    x_r = jnp.where(col == width - 1, jnp.bfloat16(0),
                    jnp.concatenate([xa[:, 1:], zc], axis=1))

    x3 = jnp.concatenate([x_l, xa, x_r], axis=0)
    z = jnp.dot(w_ref[...], x3, preferred_element_type=jnp.float32)

    ys = []
    zrow = jnp.zeros((cout, width), jnp.float32)
    for b in range(B):
        lo = b * M
        z0 = z[:cout, lo:lo + M]
        z1 = z[cout:2 * cout, lo:lo + M]
        z2 = z[2 * cout:, lo:lo + M]
        ys.append(z1 + jnp.concatenate([zrow, z0[:, :-width]], axis=1)
                     + jnp.concatenate([z2[:, width:], zrow], axis=1))
    return ys


def _accum_stats(st_sc, ys, first):
    cs = sum(jnp.sum(y, axis=1, keepdims=True) for y in ys)
    css = sum(jnp.sum(y * y, axis=1, keepdims=True) for y in ys)
    s = jnp.stack([cs, css], axis=0)

    @pl.when(first)
    def _():
        st_sc[...] = s

    @pl.when(jnp.logical_not(first))
    def _():
        st_sc[...] = st_sc[...] + s


def _bn_params(st_sc, g_ref, b_ref, count, eps):
    s = st_sc[...]
    mu = s[0] / count
    var = s[1] / count - mu * mu
    scale = g_ref[...] * jax.lax.rsqrt(var + eps)
    shift = b_ref[...] - mu * scale
    return scale, shift


def _fused_kernel(x_ref, w1_ref, w2_ref, g1_ref, b1_ref, g2_ref, b2_ref,
                  o_ref, y_sc, st1_sc, st2_sc, *,
                  B, G, cin, c1, c2, width, M, count, eps):
    i = pl.program_id(0)

    @pl.when(i < G)
    def _phase_conv1():
        x = x_ref[...].astype(jnp.bfloat16)
        ys = _conv_block(x, w1_ref, cin=cin, cout=c1, width=width, B=B, M=M)
        for b in range(B):
            y_sc[i * B + b] = ys[b].astype(jnp.bfloat16)
        _accum_stats(st1_sc, ys, i == 0)

    @pl.when(jnp.logical_and(i >= G, i < 2 * G))
    def _phase_conv2():
        j = i - G
        scale1, shift1 = _bn_params(st1_sc, g1_ref, b1_ref, count, eps)
        a = y_sc[pl.ds(j * B, B)]
        a = jnp.maximum(a.astype(jnp.float32) * scale1 + shift1,
                        0.0).astype(jnp.bfloat16)
        ys = _conv_block(a, w2_ref, cin=c1, cout=c2, width=width, B=B, M=M)
        for b in range(B):
            y_sc[j * B + b] = ys[b].astype(jnp.bfloat16)
        _accum_stats(st2_sc, ys, j == 0)

    @pl.when(i >= 2 * G)
    def _phase_bn2():
        k = i - 2 * G
        scale2, shift2 = _bn_params(st2_sc, g2_ref, b2_ref, count, eps)
        y2 = y_sc[pl.ds(k * B, B)]
        o_ref[...] = jnp.maximum(
            y2.astype(jnp.float32) * scale2 + shift2, 0.0)


def kernel(x_nchw, w1, g1, b1, w2, g2, b2, eps=1e-5):
    N, cin, H, W = x_nchw.shape
    c1 = w1.shape[-1]
    c2 = w2.shape[-1]
    M = H * W
    count = N * M
    B = 32
    G = N // B

    x = x_nchw.reshape(N, cin, M)

    wm1 = jnp.transpose(w1, (0, 3, 1, 2)).reshape(3 * c1, 3 * cin)
    wm2 = jnp.transpose(w2, (0, 3, 1, 2)).reshape(3 * c2, 3 * c1)
    wm1 = wm1.astype(jnp.bfloat16)
    wm2 = wm2.astype(jnp.bfloat16)

    body = functools.partial(_fused_kernel, B=B, G=G, cin=cin, c1=c1, c2=c2,
                             width=W, M=M, count=count, eps=eps)
    out = pl.pallas_call(
        body,
        out_shape=jax.ShapeDtypeStruct((N, c2, M), jnp.float32),
        grid=(3 * G,),
        in_specs=[
            pl.BlockSpec((B, cin, M),
                         lambda i: (jnp.minimum(i, G - 1), 0, 0)),
            pl.BlockSpec(wm1.shape, lambda i: (0, 0)),
            pl.BlockSpec(wm2.shape, lambda i: (0, 0)),
            pl.BlockSpec((c1, 1), lambda i: (0, 0)),
            pl.BlockSpec((c1, 1), lambda i: (0, 0)),
            pl.BlockSpec((c2, 1), lambda i: (0, 0)),
            pl.BlockSpec((c2, 1), lambda i: (0, 0)),
        ],
        out_specs=pl.BlockSpec((B, c2, M),
                               lambda i: (jnp.maximum(i - 2 * G, 0), 0, 0)),
        scratch_shapes=[
            pltpu.VMEM((N, c1, M), jnp.bfloat16),
            pltpu.VMEM((2, c1, 1), jnp.float32),
            pltpu.VMEM((2, c2, 1), jnp.float32),
        ],
        compiler_params=pltpu.CompilerParams(
            dimension_semantics=("arbitrary",),
            vmem_limit_bytes=_VMEM_LIMIT),
    )(x, wm1, wm2, g1[:, None], b1[:, None], g2[:, None], b2[:, None])

    return out.reshape(N, c2, H, W)
